# Initial kernel scaffold; baseline (speedup 1.0000x reference)
import jax
import jax.numpy as jnp
from jax.experimental import pallas as pl


def kernel(x, pad_hv, idx_identit, idx_out, b, hout, wout):
    raise NotImplementedError("write your pallas kernel here")


# speedup vs baseline: 9.4081x; 9.4081x over previous
import functools

import jax
import jax.numpy as jnp
from jax import lax
from jax.experimental import pallas as pl
from jax.experimental.pallas import tpu as pltpu
from jax.experimental.pallas import tpu_sc as plsc

NK = 11
N_OUT = 64
N_B = 4
SVAL = [25 - 5 * i - 4 for i in range(NK)]
XW = NK * 3600
XPAD = XW + 16
OROWW = 64
OBR = 56 * OROWW
OBUF = 3 * OBR
NC, NS = 2, 16
NW = NC * NS
PAIRS = (N_B * N_OUT) // NW


def _slot_meta():
    meta = []

    def chunks(wlo, whi):
        whi_eff = OROWW if whi >= 56 else whi
        out = []
        for wc in range(4):
            w0 = 16 * wc
            if w0 + 16 <= wlo or w0 >= whi_eff:
                continue
            if wlo <= w0 and w0 + 16 <= whi_eff:
                out.append((w0, None, None))
            else:
                out.append((w0, max(0, wlo - w0), min(16, whi_eff - w0)))
        return out

    for s in range(NK):
        sv = SVAL[s]
        wlo, whi = max(0, -2 - sv), min(56, 58 - sv)
        for _g in range(4):
            meta.append((0 * OBR, 0, 56, chunks(wlo, whi)))
    for s in range(NK):
        sv = SVAL[s]
        hlo, hhi = max(0, -2 - sv), min(56, 58 - sv)
        for _g in range(4):
            meta.append((1 * OBR, hlo, hhi, chunks(0, 56)))
    for _t in range(9):
        for _g in range(4):
            meta.append((2 * OBR, 0, 56, chunks(0, 56)))
    return meta


_META = _slot_meta()


def _build_bases(pad_hv, idx_identit):
    cols = []
    for half, mult in ((0, 1), (4, 60)):
        ords = []
        for g in range(4):
            sidx = (21 - pad_hv[:, half + g]) // 5
            ords.append(jnp.argsort(sidx.reshape(N_OUT, NK), axis=1))
        for s in range(NK):
            for g in range(4):
                cols.append(ords[g][:, s] * 3600 + (2 + mult * SVAL[s]))
    ordb = [jnp.argsort(idx_identit[:, g].reshape(N_OUT, NK), axis=1)[:, 2:]
            for g in range(4)]
    for t in range(9):
        dy, dx = t // 3 - 1, t % 3 - 1
        for g in range(4):
            cols.append(ordb[g][:, t] * 3600 + (2 + 60 * dy + dx))
    bases = jnp.stack(cols, axis=1).astype(jnp.int32)
    return jnp.pad(bases, ((0, 0), (0, 4)))


def _sc_compute(xflat, bases):
    mesh = plsc.VectorSubcoreMesh(core_axis_name="c", subcore_axis_name="s")

    @functools.partial(
        pl.kernel,
        out_type=jax.ShapeDtypeStruct((3 * N_B * N_OUT * OBR,), jnp.float32),
        mesh=mesh,
        scratch_types=[
            pltpu.VMEM((XPAD,), jnp.float32),
            pltpu.VMEM((OBUF,), jnp.float32),
            pltpu.VMEM((128,), jnp.int32),
        ],
    )
    def k(x_hbm, bases_hbm, out_hbm, xin, outb, bvec):
        wid = lax.axis_index("s") * NC + lax.axis_index("c")

        @pl.loop(0, PAIRS)
        def _pair(pair):
            pid = wid * PAIRS + pair
            o = lax.rem(pid, N_OUT)
            b = lax.div(pid, N_OUT)
            pltpu.sync_copy(x_hbm.at[pl.ds(b * (704 * 3600) + o * XW, XW)],
                            xin.at[pl.ds(0, XW)])
            pltpu.sync_copy(bases_hbm.at[pl.ds(o * 128, 128)], bvec)

            @pl.loop(0, OBUF // 16)
            def _zero(i):
                outb[pl.ds(i * 16, 16)] = jnp.zeros((16,), jnp.float32)

            for slot, (obase, hlo, hhi, chks) in enumerate(_META):
                grp = (slot // 16) * 16
                base = bvec[pl.ds(grp, 16)][slot - grp]

                @pl.loop(hlo, hhi)
                def _row(h, base=base, obase=obase, chks=chks):
                    a0 = base + (h + 2) * 60
                    oa0 = obase + h * OROWW
                    for (w0, mlo, mhi) in chks:
                        src = xin[pl.ds(a0 + w0, 16)]
                        if mlo is not None:
                            lanes = lax.iota(jnp.int32, 16)
                            m = (lanes >= mlo) & (lanes < mhi)
                            src = jnp.where(m, src, 0.0)
                        plsc.addupdate(outb.at[pl.ds(oa0 + w0, 16)], src)

            for br in range(3):
                dst = ((br * N_B + b) * N_OUT + o) * OBR
                pltpu.sync_copy(outb.at[pl.ds(br * OBR, OBR)],
                                out_hbm.at[pl.ds(dst, OBR)])

    return k(xflat, bases)


def kernel(x, pad_hv, idx_identit, idx_out, b, hout, wout):
    del idx_out, b, hout, wout
    B_, C, H, W = x.shape
    xp = jnp.pad(x, ((0, 0), (0, 0), (2, 2), (2, 2)), mode='edge')
    xflat = xp.reshape(B_ * C * 3600)
    bases = _build_bases(pad_hv, idx_identit).reshape(-1)
    out = _sc_compute(xflat, bases)
    out = out.reshape(3, B_, N_OUT, 56, OROWW)[..., :56]
    return (out[0], out[1], out[2])


# speedup vs baseline: 19.9772x; 2.1234x over previous
import functools

import jax
import jax.numpy as jnp
from jax import lax
from jax.experimental import pallas as pl
from jax.experimental.pallas import tpu as pltpu
from jax.experimental.pallas import tpu_sc as plsc

NK = 11
N_OUT = 64
N_B = 4
SVAL = [25 - 5 * i - 4 for i in range(NK)]
XW = NK * 3600
GLO = 1632
GHI = 1152
XPAD = GLO + XW + GHI
OROWW = 64
OBR = 56 * OROWW
NC, NS = 2, 16
NW = NC * NS
PAIRS = (N_B * N_OUT) // NW


def _chunks(wlo, whi):
    whi_eff = OROWW if whi >= 56 else whi
    out = []
    for wc in range(4):
        w0 = 16 * wc
        if w0 + 16 <= wlo or w0 >= whi_eff:
            continue
        if wlo <= w0 and w0 + 16 <= whi_eff:
            out.append((w0, None, None))
        else:
            out.append((w0, max(0, wlo - w0), min(16, whi_eff - w0)))
    return out


_H_CHUNKS = [_chunks(max(0, -2 - sv), min(56, 58 - sv)) for sv in SVAL]
_V_HRANGE = [(max(0, -2 - sv), min(56, 58 - sv)) for sv in SVAL]


def _build_bases(pad_hv, idx_identit):
    cols = []
    for half, mult in ((0, 1), (4, 60)):
        ords = []
        for g in range(4):
            sidx = (21 - pad_hv[:, half + g]) // 5
            ords.append(jnp.argsort(sidx.reshape(N_OUT, NK), axis=1))
        for s in range(NK):
            for g in range(4):
                cols.append(ords[g][:, s] * 3600 + (2 + mult * SVAL[s]))
    ordb = [jnp.argsort(idx_identit[:, g].reshape(N_OUT, NK), axis=1)[:, 2:]
            for g in range(4)]
    for t in range(9):
        dy, dx = t // 3 - 1, t % 3 - 1
        for g in range(4):
            cols.append(ordb[g][:, t] * 3600 + (2 + 60 * dy + dx))
    bases = jnp.stack(cols, axis=1).astype(jnp.int32) + GLO
    return jnp.pad(bases, ((0, 0), (0, 4))).reshape(-1)


def _sc_compute(xflat, bases):
    mesh = plsc.VectorSubcoreMesh(core_axis_name="c", subcore_axis_name="s")

    @functools.partial(
        pl.kernel,
        out_type=jax.ShapeDtypeStruct((3 * N_B * N_OUT * OBR,), jnp.float32),
        mesh=mesh,
        scratch_types=[
            pltpu.VMEM((XPAD,), jnp.float32),
            pltpu.VMEM((3 * OBR,), jnp.float32),
            pltpu.VMEM((128,), jnp.int32),
        ],
    )
    def k(x_hbm, bases_hbm, out_hbm, xin, outb, bvec):
        wid = lax.axis_index("s") * NC + lax.axis_index("c")

        @pl.loop(0, PAIRS)
        def _pair(pair):
            pid = wid * PAIRS + pair
            o = lax.rem(pid, N_OUT)
            b = lax.div(pid, N_OUT)
            pltpu.sync_copy(x_hbm.at[pl.ds(b * (704 * 3600) + o * XW, XW)],
                            xin.at[pl.ds(GLO, XW)])
            pltpu.sync_copy(bases_hbm.at[pl.ds(o * 128, 128)], bvec)

            blk = [bvec[pl.ds(16 * i, 16)] for i in range(8)]
            base = [blk[i // 16][i % 16] for i in range(124)]
            lanes = lax.iota(jnp.int32, 16)
            hmask = {}
            for s in range(NK):
                for (w0, mlo, mhi) in _H_CHUNKS[s]:
                    if mlo is not None:
                        hmask[(s, w0)] = (lanes >= mlo) & (lanes < mhi)

            @pl.loop(0, 56)
            def _row(h):
                rowoff = (h + 2) * 60
                for wc in range(4):
                    w0 = 16 * wc
                    acch = jnp.zeros((16,), jnp.float32)
                    for s in range(NK):
                        for (cw0, mlo, _mhi) in _H_CHUNKS[s]:
                            if cw0 != w0:
                                continue
                            bs = base[4 * s:4 * s + 4]
                            t = (xin[pl.ds(bs[0] + rowoff + w0, 16)]
                                 + xin[pl.ds(bs[1] + rowoff + w0, 16)]
                                 + xin[pl.ds(bs[2] + rowoff + w0, 16)]
                                 + xin[pl.ds(bs[3] + rowoff + w0, 16)])
                            if mlo is not None:
                                t = jnp.where(hmask[(s, w0)], t, 0.0)
                            acch = acch + t
                    outb[pl.ds(h * OROWW + w0, 16)] = acch
                    accv = jnp.zeros((16,), jnp.float32)
                    for s in range(NK):
                        bs = base[44 + 4 * s:44 + 4 * s + 4]
                        t = (xin[pl.ds(bs[0] + rowoff + w0, 16)]
                             + xin[pl.ds(bs[1] + rowoff + w0, 16)]
                             + xin[pl.ds(bs[2] + rowoff + w0, 16)]
                             + xin[pl.ds(bs[3] + rowoff + w0, 16)])
                        hlo, hhi = _V_HRANGE[s]
                        if hlo > 0 or hhi < 56:
                            ok = jnp.logical_and(h >= hlo, h < hhi)
                            t = jnp.where(ok, t, 0.0)
                        accv = accv + t
                    outb[pl.ds(OBR + h * OROWW + w0, 16)] = accv
                    accb = jnp.zeros((16,), jnp.float32)
                    for tt in range(9):
                        bs = base[88 + 4 * tt:88 + 4 * tt + 4]
                        accb = accb + (xin[pl.ds(bs[0] + rowoff + w0, 16)]
                                       + xin[pl.ds(bs[1] + rowoff + w0, 16)]
                                       + xin[pl.ds(bs[2] + rowoff + w0, 16)]
                                       + xin[pl.ds(bs[3] + rowoff + w0, 16)])
                    outb[pl.ds(2 * OBR + h * OROWW + w0, 16)] = accb

            for br in range(3):
                dst = ((br * N_B + b) * N_OUT + o) * OBR
                pltpu.sync_copy(outb.at[pl.ds(br * OBR, OBR)],
                                out_hbm.at[pl.ds(dst, OBR)])

    return k(xflat, bases)


def kernel(x, pad_hv, idx_identit, idx_out, b, hout, wout):
    del idx_out, b, hout, wout
    B_, C, H, W = x.shape
    xp = jnp.pad(x, ((0, 0), (0, 0), (2, 2), (2, 2)), mode='edge')
    xflat = xp.reshape(B_ * C * 3600)
    bases = _build_bases(pad_hv, idx_identit)
    out = _sc_compute(xflat, bases)
    out = out.reshape(3, B_, N_OUT, 56, OROWW)[..., :56]
    return (out[0], out[1], out[2])


# speedup vs baseline: 22.3775x; 1.1202x over previous
import functools

import jax
import jax.numpy as jnp
from jax import lax
from jax.experimental import pallas as pl
from jax.experimental.pallas import tpu as pltpu
from jax.experimental.pallas import tpu_sc as plsc

NK = 11
N_OUT = 64
N_B = 4
SVAL = [25 - 5 * i - 4 for i in range(NK)]
XW = NK * 3600
GLO = 1632
GHI = 1152
XPAD = GLO + XW + GHI
OBR = 56 * 56
OPAD = OBR + 16
NC, NS = 2, 16
NW = NC * NS
PAIRS = (N_B * N_OUT) // NW


def _chunks(wlo, whi):
    whi_eff = 64 if whi >= 56 else whi
    out = []
    for wc in range(4):
        w0 = 16 * wc
        if w0 + 16 <= wlo or w0 >= whi_eff:
            continue
        if wlo <= w0 and w0 + 16 <= whi_eff:
            out.append((w0, None, None))
        else:
            out.append((w0, max(0, wlo - w0), min(16, whi_eff - w0)))
    return out


_H_CHUNKS = [_chunks(max(0, -2 - sv), min(56, 58 - sv)) for sv in SVAL]
_V_HRANGE = [(max(0, -2 - sv), min(56, 58 - sv)) for sv in SVAL]


def _build_bases(pad_hv, idx_identit):
    cols = []
    for half, mult in ((0, 1), (4, 60)):
        ords = []
        for g in range(4):
            sidx = (21 - pad_hv[:, half + g]) // 5
            ords.append(jnp.argsort(sidx.reshape(N_OUT, NK), axis=1))
        for s in range(NK):
            for g in range(4):
                cols.append(ords[g][:, s] * 3600 + (2 + mult * SVAL[s]))
    ordb = [jnp.argsort(idx_identit[:, g].reshape(N_OUT, NK), axis=1)[:, 2:]
            for g in range(4)]
    for t in range(9):
        dy, dx = t // 3 - 1, t % 3 - 1
        for g in range(4):
            cols.append(ordb[g][:, t] * 3600 + (2 + 60 * dy + dx))
    bases = jnp.stack(cols, axis=1).astype(jnp.int32) + GLO
    return jnp.pad(bases, ((0, 0), (0, 4))).reshape(-1)


def _sc_compute(xflat, bases):
    mesh = plsc.VectorSubcoreMesh(core_axis_name="c", subcore_axis_name="s")

    @functools.partial(
        pl.kernel,
        out_type=jax.ShapeDtypeStruct((3 * N_B * N_OUT * OBR,), jnp.float32),
        mesh=mesh,
        scratch_types=[
            pltpu.VMEM((XPAD,), jnp.float32),
            pltpu.VMEM((3 * OPAD,), jnp.float32),
            pltpu.VMEM((128,), jnp.int32),
            pltpu.SMEM((128,), jnp.int32),
        ],
    )
    def k(x_hbm, bases_hbm, out_hbm, xin, outb, bvec, sbase):
        wid = lax.axis_index("s") * NC + lax.axis_index("c")

        @pl.loop(0, PAIRS)
        def _pair(pair):
            pid = wid * PAIRS + pair
            o = lax.rem(pid, N_OUT)
            b = lax.div(pid, N_OUT)
            pltpu.sync_copy(x_hbm.at[pl.ds(b * (704 * 3600) + o * XW, XW)],
                            xin.at[pl.ds(GLO, XW)])
            pltpu.sync_copy(bases_hbm.at[pl.ds(o * 128, 128)], bvec)

            blk = [bvec[pl.ds(16 * i, 16)] for i in range(8)]
            for i in range(124):
                sbase[i] = blk[i // 16][i % 16]
            lanes = lax.iota(jnp.int32, 16)
            hmask = {}
            for s in range(NK):
                for (w0, mlo, mhi) in _H_CHUNKS[s]:
                    if mlo is not None:
                        hmask[(s, w0)] = (lanes >= mlo) & (lanes < mhi)

            def ld4(slot0, off):
                def g(i):
                    return xin[pl.ds(sbase[slot0 + i] + off, 16)]
                return (g(0) + g(1)) + (g(2) + g(3))

            @pl.loop(0, 56)
            def _rowh(h):
                rowoff = (h + 2) * 60
                for wc in range(4):
                    w0 = 16 * wc
                    acch = jnp.zeros((16,), jnp.float32)
                    for s in range(NK):
                        for (cw0, mlo, _mhi) in _H_CHUNKS[s]:
                            if cw0 != w0:
                                continue
                            t = ld4(4 * s, rowoff + w0)
                            if mlo is not None:
                                t = jnp.where(hmask[(s, w0)], t, 0.0)
                            acch = acch + t
                    outb[pl.ds(h * 56 + w0, 16)] = acch

            @pl.loop(0, 56)
            def _rowv(h):
                rowoff = (h + 2) * 60
                for wc in range(4):
                    w0 = 16 * wc
                    accv = jnp.zeros((16,), jnp.float32)
                    for s in range(NK):
                        t = ld4(44 + 4 * s, rowoff + w0)
                        hlo, hhi = _V_HRANGE[s]
                        if hlo > 0 or hhi < 56:
                            ok = jnp.logical_and(h >= hlo, h < hhi)
                            t = jnp.where(ok, t, 0.0)
                        accv = accv + t
                    outb[pl.ds(OPAD + h * 56 + w0, 16)] = accv

            @pl.loop(0, 56)
            def _rowb(h):
                rowoff = (h + 2) * 60
                for wc in range(4):
                    w0 = 16 * wc
                    accb = jnp.zeros((16,), jnp.float32)
                    for tt in range(9):
                        accb = accb + ld4(88 + 4 * tt, rowoff + w0)
                    outb[pl.ds(2 * OPAD + h * 56 + w0, 16)] = accb

            for br in range(3):
                dst = ((br * N_B + b) * N_OUT + o) * OBR
                pltpu.sync_copy(outb.at[pl.ds(br * OPAD, OBR)],
                                out_hbm.at[pl.ds(dst, OBR)])

    return k(xflat, bases)


def kernel(x, pad_hv, idx_identit, idx_out, b, hout, wout):
    del idx_out, b, hout, wout
    B_, C, H, W = x.shape
    xp = jnp.pad(x, ((0, 0), (0, 0), (2, 2), (2, 2)), mode='edge')
    xflat = xp.reshape(B_ * C * 3600)
    bases = _build_bases(pad_hv, idx_identit)
    out = _sc_compute(xflat, bases)
    out = out.reshape(3, B_, N_OUT, 56, 56)
    return (out[0], out[1], out[2])
